# baseline (speedup 1.0000x reference)
import jax
import jax.numpy as jnp
from jax import lax
from jax.experimental import pallas as pl
from jax.experimental.pallas import tpu as pltpu
from jax.experimental.pallas import tpu_sc as plsc

N = 10000
D = 128
NC, NS, L = 2, 16, 16
NW = NC * NS
CH = 80
NCHUNK = 125
SLAB = 632
NPAD = NS * SLAB
DEN_PAD = 640


def _proj_out(h, asrc, adst, h_ref, as_ref, ad_ref, amax_ref):
    h_ref[...] = h
    a2 = jnp.concatenate([asrc[:, None], adst[:, None]], axis=1)
    asad = jnp.dot(h, a2, preferred_element_type=jnp.float32)
    as_ = asad[:, 0:1]
    as_ref[...] = as_
    ad_ref[...] = asad[:, 1:2]
    amax_ref[...] = jnp.full((L,), jnp.max(as_), jnp.float32)


def _proj_body(x_ref, w_ref, asrc_ref, adst_ref, h_ref, as_ref, ad_ref,
               amax_ref):
    h = jnp.dot(x_ref[...], w_ref[...], preferred_element_type=jnp.float32)
    _proj_out(h, asrc_ref[...], adst_ref[...], h_ref, as_ref, ad_ref,
              amax_ref)


_proj = pl.pallas_call(
    _proj_body,
    out_shape=[
        jax.ShapeDtypeStruct((N, D), jnp.float32),
        jax.ShapeDtypeStruct((N, 1), jnp.float32),
        jax.ShapeDtypeStruct((N, 1), jnp.float32),
        jax.ShapeDtypeStruct((L,), jnp.float32),
    ],
)


def _combine(num, den, h, as_, ad_, b):
    asmax = jnp.max(as_)
    sa = as_ + ad_
    al = jnp.maximum(sa, 0.2 * sa)
    m0 = asmax + ad_
    m = jnp.maximum(m0, 0.2 * m0)
    selfea = jnp.exp(al - m)
    dtot = den[0, :N] + den[1, :N] + selfea + 1e-16
    numt = num[0, :N] + num[1, :N] + selfea[:, None] * h
    return numt / dtot[:, None] + b


def _comb_proj_body(num_ref, den_ref, h_ref, as_ref, ad_ref, b_ref,
                    w_ref, asrc_ref, adst_ref, h2_ref, as2_ref, ad2_ref,
                    amax2_ref):
    out = _combine(num_ref[...], den_ref[...], h_ref[...], as_ref[...],
                   ad_ref[...], b_ref[...])
    x2 = jnp.maximum(out, 0.01 * out)
    h2 = jnp.dot(x2, w_ref[...], preferred_element_type=jnp.float32)
    _proj_out(h2, asrc_ref[...], adst_ref[...], h2_ref, as2_ref, ad2_ref,
              amax2_ref)


_comb_proj = pl.pallas_call(
    _comb_proj_body,
    out_shape=[
        jax.ShapeDtypeStruct((N, D), jnp.float32),
        jax.ShapeDtypeStruct((N, 1), jnp.float32),
        jax.ShapeDtypeStruct((N, 1), jnp.float32),
        jax.ShapeDtypeStruct((L,), jnp.float32),
    ],
)


def _final_body(num_ref, den_ref, h_ref, as_ref, ad_ref, b_ref, out_ref):
    out_ref[...] = _combine(num_ref[...], den_ref[...], h_ref[...],
                            as_ref[...], ad_ref[...], b_ref[...])


_final = pl.pallas_call(
    _final_body,
    out_shape=[jax.ShapeDtypeStruct((N, D), jnp.float32)],
)


def _sc_body(h_hbm, as_hbm, ad_hbm, amax_hbm, sd_hbm,
             num_hbm, den_hbm,
             amax_v, idx3, asb, adb, ea3, rows0, rows1, rows2,
             as_sh, ad_sh, out_sh, den_sh,
             isem0, isem1, isem2, asem0, asem1, asem2,
             bsem0, bsem1, bsem2, gsem0, gsem1, gsem2,
             ssem0, ssem1, ssem2, esem0, esem1, esem2):
    c = lax.axis_index("c")
    s = lax.axis_index("s")
    wid = c * NS + s

    rows = (rows0, rows1, rows2)
    isem = (isem0, isem1, isem2)
    asem = (asem0, asem1, asem2)
    bsem = (bsem0, bsem1, bsem2)
    gsem = (gsem0, gsem1, gsem2)
    ssem = (ssem0, ssem1, ssem2)
    esem = (esem0, esem1, esem2)

    @pl.when(s == 0)
    def _stage():
        pltpu.sync_copy(as_hbm, as_sh)
        pltpu.sync_copy(ad_hbm, ad_sh)
    pltpu.sync_copy(amax_hbm, amax_v)

    zf = jnp.zeros((L,), jnp.float32)

    def _zero_rows0(i, carry):
        for j in range(D // L):
            rows0[i, pl.ds(j * L, L)] = zf
        return carry
    lax.fori_loop(0, CH, _zero_rows0, 0)
    for g in range(CH // L):
        ea3[0, pl.ds(g * L, L)] = zf

    for k in range(SLAB // CH):
        pltpu.async_copy(rows0, out_sh.at[pl.ds(s * SLAB + k * CH, CH)],
                         isem0)
    rem = SLAB % CH
    pltpu.async_copy(rows0.at[pl.ds(0, rem)],
                     out_sh.at[pl.ds(s * SLAB + SLAB - rem, rem)], isem1)
    for k in range(DEN_PAD // CH):
        pltpu.async_copy(ea3.at[0],
                         den_sh.at[pl.ds(s * DEN_PAD + k * CH, CH)], isem2)
    for k in range(SLAB // CH):
        pltpu.make_async_copy(
            rows0, out_sh.at[pl.ds(s * SLAB + k * CH, CH)], isem0).wait()
    pltpu.make_async_copy(
        rows0.at[pl.ds(0, rem)],
        out_sh.at[pl.ds(s * SLAB + SLAB - rem, rem)], isem1).wait()
    for k in range(DEN_PAD // CH):
        pltpu.make_async_copy(
            ea3.at[0], den_sh.at[pl.ds(s * DEN_PAD + k * CH, CH)],
            isem2).wait()
    plsc.subcore_barrier()

    asmax = amax_v[pl.ds(0, L)][0]

    base = wid * NCHUNK

    def _start_idx(ci, b):
        pltpu.async_copy(sd_hbm.at[base + ci], idx3.at[b], isem[b])

    def _wait_idx(b):
        pltpu.make_async_copy(sd_hbm.at[0], idx3.at[b], isem[b]).wait()

    def _start_asad(b):
        pltpu.async_copy(as_sh.at[idx3.at[b, 0]], asb.at[b], asem[b])
        pltpu.async_copy(ad_sh.at[idx3.at[b, 1]], adb.at[b], bsem[b])

    def _wait_asad(b):
        pltpu.make_async_copy(as_sh.at[idx3.at[0, 0]], asb.at[b],
                              asem[b]).wait()
        pltpu.make_async_copy(ad_sh.at[idx3.at[0, 1]], adb.at[b],
                              bsem[b]).wait()

    def _start_gather(b):
        pltpu.async_copy(h_hbm.at[idx3.at[b, 0]], rows[b], gsem[b])

    def _wait_gather(b):
        pltpu.make_async_copy(h_hbm.at[idx3.at[0, 0]], rows[b],
                              gsem[b]).wait()

    def _start_scatter(b):
        pltpu.async_copy(rows[b], out_sh.at[idx3.at[b, 1]], ssem[b],
                         add=True)
        pltpu.async_copy(ea3.at[b], den_sh.at[idx3.at[b, 1]], esem[b],
                         add=True)

    def _wait_scatter(b):
        pltpu.make_async_copy(rows[b], out_sh.at[idx3.at[0, 1]],
                              ssem[b]).wait()
        pltpu.make_async_copy(ea3.at[b], den_sh.at[idx3.at[0, 1]],
                              esem[b]).wait()

    def _compute_ea(b):
        for g in range(CH // L):
            a_s = asb[b, pl.ds(g * L, L)]
            a_d = adb[b, pl.ds(g * L, L)]
            sa = a_s + a_d
            al = jnp.maximum(sa, 0.2 * sa)
            m0 = asmax + a_d
            m = jnp.maximum(m0, 0.2 * m0)
            ea3[b, pl.ds(g * L, L)] = jnp.exp(al - m)

    def _scale(b):
        rb = rows[b]

        def _body(g, carry):
            ea16 = ea3[b, pl.ds(g * L, L)]
            base_r = g * L
            for i in range(L):
                e = ea16[i]
                for j in range(D // L):
                    rb[base_r + i, pl.ds(j * L, L)] = (
                        rb[base_r + i, pl.ds(j * L, L)] * e)
            return carry
        lax.fori_loop(0, CH // L, _body, 0)

    def _phase(ci, b, first=False, fetch=True, prep=True):
        nb = (b + 1) % 3
        pb = (b + 2) % 3
        _wait_asad(b)
        _compute_ea(b)
        _wait_gather(b)
        _scale(b)
        _start_scatter(b)
        if not first:
            _wait_scatter(pb)
        if fetch:
            _start_idx(ci + 2, pb)
        if prep:
            _wait_idx(nb)
            _start_asad(nb)
            _start_gather(nb)

    _start_idx(0, 0)
    _start_idx(1, 1)
    _wait_idx(0)
    _start_asad(0)
    _start_gather(0)
    _phase(0, 0, first=True)

    def _loop_body(i, carry):
        ci = i * 3 + 1
        for p in range(3):
            _phase(ci + p, (1 + p) % 3)
        return carry

    lax.fori_loop(0, 40, _loop_body, 0)

    _phase(121, 1)
    _phase(122, 2)
    _phase(123, 0, fetch=False)
    _phase(124, 1, fetch=False, prep=False)
    _wait_scatter(1)

    plsc.subcore_barrier()
    pltpu.sync_copy(out_sh.at[pl.ds(s * SLAB, SLAB)],
                    num_hbm.at[c, pl.ds(s * SLAB, SLAB)])
    pltpu.sync_copy(den_sh.at[pl.ds(s * DEN_PAD, DEN_PAD)],
                    den_hbm.at[c, pl.ds(s * DEN_PAD, DEN_PAD)])


_sc_edge = pl.kernel(
    _sc_body,
    out_type=[
        jax.ShapeDtypeStruct((NC, NPAD, D), jnp.float32),
        jax.ShapeDtypeStruct((NC, NS * DEN_PAD), jnp.float32),
    ],
    mesh=plsc.VectorSubcoreMesh(core_axis_name="c", subcore_axis_name="s"),
    compiler_params=pltpu.CompilerParams(needs_layout_passes=False),
    scratch_types=[
        pltpu.VMEM((L,), jnp.float32),
        pltpu.VMEM((3, 2, CH), jnp.int32),
        pltpu.VMEM((3, CH), jnp.float32),
        pltpu.VMEM((3, CH), jnp.float32),
        pltpu.VMEM((3, CH), jnp.float32),
        pltpu.VMEM((CH, D), jnp.float32),
        pltpu.VMEM((CH, D), jnp.float32),
        pltpu.VMEM((CH, D), jnp.float32),
        pltpu.VMEM_SHARED((N,), jnp.float32),
        pltpu.VMEM_SHARED((N,), jnp.float32),
        pltpu.VMEM_SHARED((NPAD, D), jnp.float32),
        pltpu.VMEM_SHARED((NS * DEN_PAD,), jnp.float32),
        pltpu.SemaphoreType.DMA,
        pltpu.SemaphoreType.DMA,
        pltpu.SemaphoreType.DMA,
        pltpu.SemaphoreType.DMA,
        pltpu.SemaphoreType.DMA,
        pltpu.SemaphoreType.DMA,
        pltpu.SemaphoreType.DMA,
        pltpu.SemaphoreType.DMA,
        pltpu.SemaphoreType.DMA,
        pltpu.SemaphoreType.DMA,
        pltpu.SemaphoreType.DMA,
        pltpu.SemaphoreType.DMA,
        pltpu.SemaphoreType.DMA,
        pltpu.SemaphoreType.DMA,
        pltpu.SemaphoreType.DMA,
        pltpu.SemaphoreType.DMA,
        pltpu.SemaphoreType.DMA,
        pltpu.SemaphoreType.DMA,
    ],
)


def kernel(x, edge_index, W0, a_src0, a_dst0, b0, W1, a_src1, a_dst1, b1,
           W2, a_src2, a_dst2, b2):
    src3 = edge_index[0].reshape(NW, NCHUNK, CH)
    dst3 = edge_index[1].reshape(NW, NCHUNK, CH)
    sd = jnp.stack([src3, dst3], axis=2).reshape(NW * NCHUNK, 2, CH)

    h0, as0, ad0, am0 = _proj(x, W0, a_src0, a_dst0)
    as0, ad0 = as0.reshape(N), ad0.reshape(N)
    num0, den0 = _sc_edge(h0, as0, ad0, am0, sd)
    h1, as1, ad1, am1 = _comb_proj(num0, den0, h0, as0, ad0, b0,
                                   W1, a_src1, a_dst1)
    as1, ad1 = as1.reshape(N), ad1.reshape(N)
    num1, den1 = _sc_edge(h1, as1, ad1, am1, sd)
    h2, as2, ad2, am2 = _comb_proj(num1, den1, h1, as1, ad1, b1,
                                   W2, a_src2, a_dst2)
    as2, ad2 = as2.reshape(N), ad2.reshape(N)
    num2, den2 = _sc_edge(h2, as2, ad2, am2, sd)
    (out,) = _final(num2, den2, h2, as2, ad2, b2)
    return out


# speedup vs baseline: 1.0361x; 1.0361x over previous
import jax
import jax.numpy as jnp
from jax import lax
from jax.experimental import pallas as pl
from jax.experimental.pallas import tpu as pltpu
from jax.experimental.pallas import tpu_sc as plsc

N = 10000
D = 128
NC, NS, L = 2, 16, 16
NW = NC * NS
CH = 80
NCHUNK = 125
SLAB = 632
NPAD = NS * SLAB
DEN_PAD = 640


def _proj_body(x_ref, w_ref, asrc_ref, adst_ref, h_ref, as_ref, ad_ref,
               amax_ref):
    h = jnp.dot(x_ref[...], w_ref[...], preferred_element_type=jnp.float32)
    h_ref[...] = h
    as_ = jnp.sum(h * asrc_ref[...], axis=1)
    as_ref[...] = as_
    ad_ref[...] = jnp.sum(h * adst_ref[...], axis=1)
    amax_ref[...] = jnp.full((L,), jnp.max(as_), jnp.float32)


_proj = pl.pallas_call(
    _proj_body,
    out_shape=[
        jax.ShapeDtypeStruct((N, D), jnp.float32),
        jax.ShapeDtypeStruct((N,), jnp.float32),
        jax.ShapeDtypeStruct((N,), jnp.float32),
        jax.ShapeDtypeStruct((L,), jnp.float32),
    ],
)


def _combine(num, den, h, as_, ad_, b):
    asmax = jnp.max(as_)
    sa = as_ + ad_
    al = jnp.maximum(sa, 0.2 * sa)
    m0 = asmax + ad_
    m = jnp.maximum(m0, 0.2 * m0)
    selfea = jnp.exp(al - m)
    dtot = den[0, :N] + den[1, :N] + selfea + 1e-16
    numt = num[0, :N] + num[1, :N] + selfea[:, None] * h
    return numt / dtot[:, None] + b


def _comb_proj_body(num_ref, den_ref, h_ref, as_ref, ad_ref, b_ref,
                    w_ref, asrc_ref, adst_ref, h2_ref, as2_ref, ad2_ref,
                    amax2_ref):
    out = _combine(num_ref[...], den_ref[...], h_ref[...], as_ref[...],
                   ad_ref[...], b_ref[...])
    x2 = jnp.maximum(out, 0.01 * out)
    h2 = jnp.dot(x2, w_ref[...], preferred_element_type=jnp.float32)
    h2_ref[...] = h2
    as2 = jnp.sum(h2 * asrc_ref[...], axis=1)
    as2_ref[...] = as2
    ad2_ref[...] = jnp.sum(h2 * adst_ref[...], axis=1)
    amax2_ref[...] = jnp.full((L,), jnp.max(as2), jnp.float32)


_comb_proj = pl.pallas_call(
    _comb_proj_body,
    out_shape=[
        jax.ShapeDtypeStruct((N, D), jnp.float32),
        jax.ShapeDtypeStruct((N,), jnp.float32),
        jax.ShapeDtypeStruct((N,), jnp.float32),
        jax.ShapeDtypeStruct((L,), jnp.float32),
    ],
)


def _final_body(num_ref, den_ref, h_ref, as_ref, ad_ref, b_ref, out_ref):
    out_ref[...] = _combine(num_ref[...], den_ref[...], h_ref[...],
                            as_ref[...], ad_ref[...], b_ref[...])


_final = pl.pallas_call(
    _final_body,
    out_shape=[jax.ShapeDtypeStruct((N, D), jnp.float32)],
)


def _sc_body(h_hbm, as_hbm, ad_hbm, amax_hbm, sd_hbm,
             num_hbm, den_hbm,
             amax_v, idx3, asb, adb, ea3, rows0, rows1, rows2,
             as_sh, ad_sh, out_sh, den_sh,
             isem0, isem1, isem2, asem0, asem1, asem2,
             bsem0, bsem1, bsem2, gsem0, gsem1, gsem2,
             ssem0, ssem1, ssem2, esem0, esem1, esem2):
    c = lax.axis_index("c")
    s = lax.axis_index("s")
    wid = c * NS + s

    rows = (rows0, rows1, rows2)
    isem = (isem0, isem1, isem2)
    asem = (asem0, asem1, asem2)
    bsem = (bsem0, bsem1, bsem2)
    gsem = (gsem0, gsem1, gsem2)
    ssem = (ssem0, ssem1, ssem2)
    esem = (esem0, esem1, esem2)

    @pl.when(s == 0)
    def _stage():
        pltpu.sync_copy(as_hbm, as_sh)
        pltpu.sync_copy(ad_hbm, ad_sh)
    pltpu.sync_copy(amax_hbm, amax_v)

    zf = jnp.zeros((L,), jnp.float32)

    def _zero_rows0(i, carry):
        for j in range(D // L):
            rows0[i, pl.ds(j * L, L)] = zf
        return carry
    lax.fori_loop(0, CH, _zero_rows0, 0)
    for g in range(CH // L):
        ea3[0, pl.ds(g * L, L)] = zf

    for k in range(SLAB // CH):
        pltpu.sync_copy(rows0, out_sh.at[pl.ds(s * SLAB + k * CH, CH)])
    rem = SLAB % CH
    pltpu.sync_copy(rows0.at[pl.ds(0, rem)],
                    out_sh.at[pl.ds(s * SLAB + SLAB - rem, rem)])
    for k in range(DEN_PAD // CH):
        pltpu.sync_copy(ea3.at[0], den_sh.at[pl.ds(s * DEN_PAD + k * CH, CH)])
    plsc.subcore_barrier()

    asmax = amax_v[pl.ds(0, L)][0]

    base = wid * NCHUNK

    def _start_idx(ci, b):
        pltpu.async_copy(sd_hbm.at[base + ci], idx3.at[b], isem[b])

    def _wait_idx(b):
        pltpu.make_async_copy(sd_hbm.at[0], idx3.at[b], isem[b]).wait()

    def _start_asad(b):
        pltpu.async_copy(as_sh.at[idx3.at[b, 0]], asb.at[b], asem[b])
        pltpu.async_copy(ad_sh.at[idx3.at[b, 1]], adb.at[b], bsem[b])

    def _wait_asad(b):
        pltpu.make_async_copy(as_sh.at[idx3.at[0, 0]], asb.at[b],
                              asem[b]).wait()
        pltpu.make_async_copy(ad_sh.at[idx3.at[0, 1]], adb.at[b],
                              bsem[b]).wait()

    def _start_gather(b):
        pltpu.async_copy(h_hbm.at[idx3.at[b, 0]], rows[b], gsem[b])

    def _wait_gather(b):
        pltpu.make_async_copy(h_hbm.at[idx3.at[0, 0]], rows[b],
                              gsem[b]).wait()

    def _start_scatter(b):
        pltpu.async_copy(rows[b], out_sh.at[idx3.at[b, 1]], ssem[b],
                         add=True)
        pltpu.async_copy(ea3.at[b], den_sh.at[idx3.at[b, 1]], esem[b],
                         add=True)

    def _wait_scatter(b):
        pltpu.make_async_copy(rows[b], out_sh.at[idx3.at[0, 1]],
                              ssem[b]).wait()
        pltpu.make_async_copy(ea3.at[b], den_sh.at[idx3.at[0, 1]],
                              esem[b]).wait()

    def _compute_ea(b):
        for g in range(CH // L):
            a_s = asb[b, pl.ds(g * L, L)]
            a_d = adb[b, pl.ds(g * L, L)]
            sa = a_s + a_d
            al = jnp.maximum(sa, 0.2 * sa)
            m0 = asmax + a_d
            m = jnp.maximum(m0, 0.2 * m0)
            ea3[b, pl.ds(g * L, L)] = jnp.exp(al - m)

    def _scale(b):
        rb = rows[b]

        def _body(g, carry):
            ea16 = ea3[b, pl.ds(g * L, L)]
            base_r = g * L
            for i in range(L):
                e = ea16.at[jnp.full((L,), i, jnp.int32)].get(
                    mode="promise_in_bounds")
                for j in range(D // L):
                    rb[base_r + i, pl.ds(j * L, L)] = (
                        rb[base_r + i, pl.ds(j * L, L)] * e)
            return carry
        lax.fori_loop(0, CH // L, _body, 0)

    def _phase(ci, b, first=False, fetch=True, prep=True):
        nb = (b + 1) % 3
        pb = (b + 2) % 3
        _wait_asad(b)
        _compute_ea(b)
        _wait_gather(b)
        _scale(b)
        _start_scatter(b)
        if not first:
            _wait_scatter(pb)
        if fetch:
            _start_idx(ci + 2, pb)
        if prep:
            _wait_idx(nb)
            _start_asad(nb)
            _start_gather(nb)

    _start_idx(0, 0)
    _start_idx(1, 1)
    _wait_idx(0)
    _start_asad(0)
    _start_gather(0)
    _phase(0, 0, first=True)

    def _loop_body(i, carry):
        ci = i * 3 + 1
        for p in range(3):
            _phase(ci + p, (1 + p) % 3)
        return carry

    lax.fori_loop(0, 40, _loop_body, 0)

    _phase(121, 1)
    _phase(122, 2)
    _phase(123, 0, fetch=False)
    _phase(124, 1, fetch=False, prep=False)
    _wait_scatter(1)

    plsc.subcore_barrier()
    pltpu.sync_copy(out_sh.at[pl.ds(s * SLAB, SLAB)],
                    num_hbm.at[c, pl.ds(s * SLAB, SLAB)])
    pltpu.sync_copy(den_sh.at[pl.ds(s * DEN_PAD, DEN_PAD)],
                    den_hbm.at[c, pl.ds(s * DEN_PAD, DEN_PAD)])


_sc_edge = pl.kernel(
    _sc_body,
    out_type=[
        jax.ShapeDtypeStruct((NC, NPAD, D), jnp.float32),
        jax.ShapeDtypeStruct((NC, NS * DEN_PAD), jnp.float32),
    ],
    mesh=plsc.VectorSubcoreMesh(core_axis_name="c", subcore_axis_name="s"),
    compiler_params=pltpu.CompilerParams(needs_layout_passes=False),
    scratch_types=[
        pltpu.VMEM((L,), jnp.float32),
        pltpu.VMEM((3, 2, CH), jnp.int32),
        pltpu.VMEM((3, CH), jnp.float32),
        pltpu.VMEM((3, CH), jnp.float32),
        pltpu.VMEM((3, CH), jnp.float32),
        pltpu.VMEM((CH, D), jnp.float32),
        pltpu.VMEM((CH, D), jnp.float32),
        pltpu.VMEM((CH, D), jnp.float32),
        pltpu.VMEM_SHARED((N,), jnp.float32),
        pltpu.VMEM_SHARED((N,), jnp.float32),
        pltpu.VMEM_SHARED((NPAD, D), jnp.float32),
        pltpu.VMEM_SHARED((NS * DEN_PAD,), jnp.float32),
        pltpu.SemaphoreType.DMA,
        pltpu.SemaphoreType.DMA,
        pltpu.SemaphoreType.DMA,
        pltpu.SemaphoreType.DMA,
        pltpu.SemaphoreType.DMA,
        pltpu.SemaphoreType.DMA,
        pltpu.SemaphoreType.DMA,
        pltpu.SemaphoreType.DMA,
        pltpu.SemaphoreType.DMA,
        pltpu.SemaphoreType.DMA,
        pltpu.SemaphoreType.DMA,
        pltpu.SemaphoreType.DMA,
        pltpu.SemaphoreType.DMA,
        pltpu.SemaphoreType.DMA,
        pltpu.SemaphoreType.DMA,
        pltpu.SemaphoreType.DMA,
        pltpu.SemaphoreType.DMA,
        pltpu.SemaphoreType.DMA,
    ],
)


def kernel(x, edge_index, W0, a_src0, a_dst0, b0, W1, a_src1, a_dst1, b1,
           W2, a_src2, a_dst2, b2):
    src3 = edge_index[0].reshape(NW, NCHUNK, CH)
    dst3 = edge_index[1].reshape(NW, NCHUNK, CH)
    sd = jnp.stack([src3, dst3], axis=2).reshape(NW * NCHUNK, 2, CH)

    h0, as0, ad0, am0 = _proj(x, W0, a_src0, a_dst0)
    num0, den0 = _sc_edge(h0, as0, ad0, am0, sd)
    h1, as1, ad1, am1 = _comb_proj(num0, den0, h0, as0, ad0, b0,
                                   W1, a_src1, a_dst1)
    num1, den1 = _sc_edge(h1, as1, ad1, am1, sd)
    h2, as2, ad2, am2 = _comb_proj(num1, den1, h1, as1, ad1, b1,
                                   W2, a_src2, a_dst2)
    num2, den2 = _sc_edge(h2, as2, ad2, am2, sd)
    (out,) = _final(num2, den2, h2, as2, ad2, b2)
    return out


# speedup vs baseline: 1.2894x; 1.2445x over previous
import jax
import jax.numpy as jnp
from jax import lax
from jax.experimental import pallas as pl
from jax.experimental.pallas import tpu as pltpu
from jax.experimental.pallas import tpu_sc as plsc

N = 10000
D = 128
NC, NS, L = 2, 16, 16
NW = NC * NS
CH = 80
NCHUNK = 125
SLAB = 632
NPAD = NS * SLAB
DEN_PAD = 640


def _proj_body(x_ref, w_ref, asrc_ref, adst_ref, h_ref, as_ref, ad_ref,
               amax_ref):
    h = jnp.dot(x_ref[...], w_ref[...], preferred_element_type=jnp.float32)
    h_ref[...] = h
    as_ = jnp.sum(h * asrc_ref[...], axis=1)
    as_ref[...] = as_
    ad_ref[...] = jnp.sum(h * adst_ref[...], axis=1)
    amax_ref[...] = jnp.full((L,), jnp.max(as_), jnp.float32)


_proj = pl.pallas_call(
    _proj_body,
    out_shape=[
        jax.ShapeDtypeStruct((N, D), jnp.float32),
        jax.ShapeDtypeStruct((N,), jnp.float32),
        jax.ShapeDtypeStruct((N,), jnp.float32),
        jax.ShapeDtypeStruct((L,), jnp.float32),
    ],
)


def _combine(num, den, h, as_, ad_, b):
    asmax = jnp.max(as_)
    sa = as_ + ad_
    al = jnp.maximum(sa, 0.2 * sa)
    m0 = asmax + ad_
    m = jnp.maximum(m0, 0.2 * m0)
    selfea = jnp.exp(al - m)
    dtot = den[0, :N] + den[1, :N] + selfea + 1e-16
    numt = num[0, :N] + num[1, :N] + selfea[:, None] * h
    return numt / dtot[:, None] + b


def _comb_proj_body(num_ref, den_ref, h_ref, as_ref, ad_ref, b_ref,
                    w_ref, asrc_ref, adst_ref, h2_ref, as2_ref, ad2_ref,
                    amax2_ref):
    out = _combine(num_ref[...], den_ref[...], h_ref[...], as_ref[...],
                   ad_ref[...], b_ref[...])
    x2 = jnp.maximum(out, 0.01 * out)
    h2 = jnp.dot(x2, w_ref[...], preferred_element_type=jnp.float32)
    h2_ref[...] = h2
    as2 = jnp.sum(h2 * asrc_ref[...], axis=1)
    as2_ref[...] = as2
    ad2_ref[...] = jnp.sum(h2 * adst_ref[...], axis=1)
    amax2_ref[...] = jnp.full((L,), jnp.max(as2), jnp.float32)


_comb_proj = pl.pallas_call(
    _comb_proj_body,
    out_shape=[
        jax.ShapeDtypeStruct((N, D), jnp.float32),
        jax.ShapeDtypeStruct((N,), jnp.float32),
        jax.ShapeDtypeStruct((N,), jnp.float32),
        jax.ShapeDtypeStruct((L,), jnp.float32),
    ],
)


def _final_body(num_ref, den_ref, h_ref, as_ref, ad_ref, b_ref, out_ref):
    out_ref[...] = _combine(num_ref[...], den_ref[...], h_ref[...],
                            as_ref[...], ad_ref[...], b_ref[...])


_final = pl.pallas_call(
    _final_body,
    out_shape=[jax.ShapeDtypeStruct((N, D), jnp.float32)],
)


def _sc_body(h_hbm, as_hbm, ad_hbm, amax_hbm, sd_hbm,
             num_hbm, den_hbm,
             amax_v, idx3, asb, adb, ea3, rows0, rows1, rows2,
             as_sh, ad_sh, out_sh, den_sh,
             isem0, isem1, isem2, asem0, asem1, asem2,
             bsem0, bsem1, bsem2, gsem0, gsem1, gsem2,
             ssem0, ssem1, ssem2, esem0, esem1, esem2):
    c = lax.axis_index("c")
    s = lax.axis_index("s")
    wid = c * NS + s

    rows = (rows0, rows1, rows2)
    isem = (isem0, isem1, isem2)
    asem = (asem0, asem1, asem2)
    bsem = (bsem0, bsem1, bsem2)
    gsem = (gsem0, gsem1, gsem2)
    ssem = (ssem0, ssem1, ssem2)
    esem = (esem0, esem1, esem2)

    @pl.when(s == 0)
    def _stage():
        pltpu.sync_copy(as_hbm, as_sh)
        pltpu.sync_copy(ad_hbm, ad_sh)
    pltpu.sync_copy(amax_hbm, amax_v)

    zf = jnp.zeros((L,), jnp.float32)

    def _zero_rows0(i, carry):
        for j in range(D // L):
            rows0[i, pl.ds(j * L, L)] = zf
        return carry
    lax.fori_loop(0, CH, _zero_rows0, 0)
    for g in range(CH // L):
        ea3[0, pl.ds(g * L, L)] = zf

    for k in range(SLAB // CH):
        pltpu.sync_copy(rows0, out_sh.at[pl.ds(s * SLAB + k * CH, CH)])
    rem = SLAB % CH
    pltpu.sync_copy(rows0.at[pl.ds(0, rem)],
                    out_sh.at[pl.ds(s * SLAB + SLAB - rem, rem)])
    for k in range(DEN_PAD // CH):
        pltpu.sync_copy(ea3.at[0], den_sh.at[pl.ds(s * DEN_PAD + k * CH, CH)])
    plsc.subcore_barrier()

    asmax = amax_v[pl.ds(0, L)][0]

    base = wid * NCHUNK

    def _start_idx(ci, b):
        pltpu.async_copy(sd_hbm.at[base + ci], idx3.at[b], isem[b])

    def _wait_idx(b):
        pltpu.make_async_copy(sd_hbm.at[0], idx3.at[b], isem[b]).wait()

    def _start_asad(b):
        pltpu.async_copy(as_sh.at[idx3.at[b, 0]], asb.at[b], asem[b])
        pltpu.async_copy(ad_sh.at[idx3.at[b, 1]], adb.at[b], bsem[b])

    def _wait_asad(b):
        pltpu.make_async_copy(as_sh.at[idx3.at[0, 0]], asb.at[b],
                              asem[b]).wait()
        pltpu.make_async_copy(ad_sh.at[idx3.at[0, 1]], adb.at[b],
                              bsem[b]).wait()

    def _start_gather(b):
        pltpu.async_copy(h_hbm.at[idx3.at[b, 0]], rows[b], gsem[b])

    def _wait_gather(b):
        pltpu.make_async_copy(h_hbm.at[idx3.at[0, 0]], rows[b],
                              gsem[b]).wait()

    def _start_scatter(b):
        pltpu.async_copy(rows[b], out_sh.at[idx3.at[b, 1]], ssem[b],
                         add=True)
        pltpu.async_copy(ea3.at[b], den_sh.at[idx3.at[b, 1]], esem[b],
                         add=True)

    def _wait_scatter(b):
        pltpu.make_async_copy(rows[b], out_sh.at[idx3.at[0, 1]],
                              ssem[b]).wait()
        pltpu.make_async_copy(ea3.at[b], den_sh.at[idx3.at[0, 1]],
                              esem[b]).wait()

    def _compute_ea(b):
        for g in range(CH // L):
            a_s = asb[b, pl.ds(g * L, L)]
            a_d = adb[b, pl.ds(g * L, L)]
            sa = a_s + a_d
            al = jnp.maximum(sa, 0.2 * sa)
            m0 = asmax + a_d
            m = jnp.maximum(m0, 0.2 * m0)
            ea3[b, pl.ds(g * L, L)] = jnp.exp(al - m)

    def _scale(b):
        rb = rows[b]

        def _body(g, carry):
            ea16 = ea3[b, pl.ds(g * L, L)]
            base_r = g * L
            for i in range(L):
                e = ea16.at[jnp.full((L,), i, jnp.int32)].get(
                    mode="promise_in_bounds")
                for j in range(D // L):
                    rb[base_r + i, pl.ds(j * L, L)] = (
                        rb[base_r + i, pl.ds(j * L, L)] * e)
            return carry
        lax.fori_loop(0, CH // L, _body, 0)

    def _phase(ci, b, first=False, fetch=True, prep=True):
        nb = (b + 1) % 3
        pb = (b + 2) % 3
        _wait_asad(b)
        _compute_ea(b)
        _wait_gather(b)
        if prep:
            _wait_idx(nb)
            _start_asad(nb)
            _start_gather(nb)
        _scale(b)
        _start_scatter(b)
        if not first:
            _wait_scatter(pb)
        if fetch:
            _start_idx(ci + 2, pb)

    _start_idx(0, 0)
    _start_idx(1, 1)
    _wait_idx(0)
    _start_asad(0)
    _start_gather(0)
    _phase(0, 0, first=True)

    def _loop_body(i, carry):
        ci = i * 3 + 1
        for p in range(3):
            _phase(ci + p, (1 + p) % 3)
        return carry

    lax.fori_loop(0, 40, _loop_body, 0)

    _phase(121, 1)
    _phase(122, 2)
    _phase(123, 0, fetch=False)
    _phase(124, 1, fetch=False, prep=False)
    _wait_scatter(1)

    plsc.subcore_barrier()
    pltpu.sync_copy(out_sh.at[pl.ds(s * SLAB, SLAB)],
                    num_hbm.at[c, pl.ds(s * SLAB, SLAB)])
    pltpu.sync_copy(den_sh.at[pl.ds(s * DEN_PAD, DEN_PAD)],
                    den_hbm.at[c, pl.ds(s * DEN_PAD, DEN_PAD)])


_sc_edge = pl.kernel(
    _sc_body,
    out_type=[
        jax.ShapeDtypeStruct((NC, NPAD, D), jnp.float32),
        jax.ShapeDtypeStruct((NC, NS * DEN_PAD), jnp.float32),
    ],
    mesh=plsc.VectorSubcoreMesh(core_axis_name="c", subcore_axis_name="s"),
    compiler_params=pltpu.CompilerParams(needs_layout_passes=False),
    scratch_types=[
        pltpu.VMEM((L,), jnp.float32),
        pltpu.VMEM((3, 2, CH), jnp.int32),
        pltpu.VMEM((3, CH), jnp.float32),
        pltpu.VMEM((3, CH), jnp.float32),
        pltpu.VMEM((3, CH), jnp.float32),
        pltpu.VMEM((CH, D), jnp.float32),
        pltpu.VMEM((CH, D), jnp.float32),
        pltpu.VMEM((CH, D), jnp.float32),
        pltpu.VMEM_SHARED((N,), jnp.float32),
        pltpu.VMEM_SHARED((N,), jnp.float32),
        pltpu.VMEM_SHARED((NPAD, D), jnp.float32),
        pltpu.VMEM_SHARED((NS * DEN_PAD,), jnp.float32),
        pltpu.SemaphoreType.DMA,
        pltpu.SemaphoreType.DMA,
        pltpu.SemaphoreType.DMA,
        pltpu.SemaphoreType.DMA,
        pltpu.SemaphoreType.DMA,
        pltpu.SemaphoreType.DMA,
        pltpu.SemaphoreType.DMA,
        pltpu.SemaphoreType.DMA,
        pltpu.SemaphoreType.DMA,
        pltpu.SemaphoreType.DMA,
        pltpu.SemaphoreType.DMA,
        pltpu.SemaphoreType.DMA,
        pltpu.SemaphoreType.DMA,
        pltpu.SemaphoreType.DMA,
        pltpu.SemaphoreType.DMA,
        pltpu.SemaphoreType.DMA,
        pltpu.SemaphoreType.DMA,
        pltpu.SemaphoreType.DMA,
    ],
)


def kernel(x, edge_index, W0, a_src0, a_dst0, b0, W1, a_src1, a_dst1, b1,
           W2, a_src2, a_dst2, b2):
    src3 = edge_index[0].reshape(NW, NCHUNK, CH)
    dst3 = edge_index[1].reshape(NW, NCHUNK, CH)
    sd = jnp.stack([src3, dst3], axis=2).reshape(NW * NCHUNK, 2, CH)

    h0, as0, ad0, am0 = _proj(x, W0, a_src0, a_dst0)
    num0, den0 = _sc_edge(h0, as0, ad0, am0, sd)
    h1, as1, ad1, am1 = _comb_proj(num0, den0, h0, as0, ad0, b0,
                                   W1, a_src1, a_dst1)
    num1, den1 = _sc_edge(h1, as1, ad1, am1, sd)
    h2, as2, ad2, am2 = _comb_proj(num1, den1, h1, as1, ad1, b1,
                                   W2, a_src2, a_dst2)
    num2, den2 = _sc_edge(h2, as2, ad2, am2, sd)
    (out,) = _final(num2, den2, h2, as2, ad2, b2)
    return out


# speedup vs baseline: 1.2967x; 1.0057x over previous
import jax
import jax.numpy as jnp
from jax import lax
from jax.experimental import pallas as pl
from jax.experimental.pallas import tpu as pltpu
from jax.experimental.pallas import tpu_sc as plsc

N = 10000
D = 128
NC, NS, L = 2, 16, 16
NW = NC * NS
CH = 80
NCHUNK = 125
SLAB = 632
NPAD = NS * SLAB
DEN_PAD = 640


def _proj_body(x_ref, w_ref, asrc_ref, adst_ref, h_ref, as_ref, ad_ref,
               amax_ref):
    h = jnp.dot(x_ref[...], w_ref[...], preferred_element_type=jnp.float32)
    h_ref[...] = h
    as_ = jnp.sum(h * asrc_ref[...], axis=1)
    as_ref[...] = as_
    ad_ref[...] = jnp.sum(h * adst_ref[...], axis=1)
    amax_ref[...] = jnp.full((L,), jnp.max(as_), jnp.float32)


_proj = pl.pallas_call(
    _proj_body,
    out_shape=[
        jax.ShapeDtypeStruct((N, D), jnp.float32),
        jax.ShapeDtypeStruct((N,), jnp.float32),
        jax.ShapeDtypeStruct((N,), jnp.float32),
        jax.ShapeDtypeStruct((L,), jnp.float32),
    ],
)


def _combine(num, den, h, as_, ad_, b):
    asmax = jnp.max(as_)
    sa = as_ + ad_
    al = jnp.maximum(sa, 0.2 * sa)
    m0 = asmax + ad_
    m = jnp.maximum(m0, 0.2 * m0)
    selfea = jnp.exp(al - m)
    dtot = den[0, :N] + den[1, :N] + selfea + 1e-16
    numt = num[0, :N] + num[1, :N] + selfea[:, None] * h
    return numt / dtot[:, None] + b


def _comb_proj_body(num_ref, den_ref, h_ref, as_ref, ad_ref, b_ref,
                    w_ref, asrc_ref, adst_ref, h2_ref, as2_ref, ad2_ref,
                    amax2_ref):
    out = _combine(num_ref[...], den_ref[...], h_ref[...], as_ref[...],
                   ad_ref[...], b_ref[...])
    x2 = jnp.maximum(out, 0.01 * out)
    h2 = jnp.dot(x2, w_ref[...], preferred_element_type=jnp.float32)
    h2_ref[...] = h2
    as2 = jnp.sum(h2 * asrc_ref[...], axis=1)
    as2_ref[...] = as2
    ad2_ref[...] = jnp.sum(h2 * adst_ref[...], axis=1)
    amax2_ref[...] = jnp.full((L,), jnp.max(as2), jnp.float32)


_comb_proj = pl.pallas_call(
    _comb_proj_body,
    out_shape=[
        jax.ShapeDtypeStruct((N, D), jnp.float32),
        jax.ShapeDtypeStruct((N,), jnp.float32),
        jax.ShapeDtypeStruct((N,), jnp.float32),
        jax.ShapeDtypeStruct((L,), jnp.float32),
    ],
)


def _final_body(num_ref, den_ref, h_ref, as_ref, ad_ref, b_ref, out_ref):
    out_ref[...] = _combine(num_ref[...], den_ref[...], h_ref[...],
                            as_ref[...], ad_ref[...], b_ref[...])


_final = pl.pallas_call(
    _final_body,
    out_shape=[jax.ShapeDtypeStruct((N, D), jnp.float32)],
)


def _sc_body(h_hbm, as_hbm, ad_hbm, amax_hbm, sd_hbm,
             num_hbm, den_hbm,
             amax_v, idx3, asb, adb, ea3, rows0, rows1, rows2,
             as_sh, ad_sh, out_sh, den_sh,
             isem0, isem1, isem2, asem0, asem1, asem2,
             bsem0, bsem1, bsem2, gsem0, gsem1, gsem2,
             ssem0, ssem1, ssem2, esem0, esem1, esem2):
    c = lax.axis_index("c")
    s = lax.axis_index("s")
    wid = c * NS + s

    rows = (rows0, rows1, rows2)
    isem = (isem0, isem1, isem2)
    asem = (asem0, asem1, asem2)
    bsem = (bsem0, bsem1, bsem2)
    gsem = (gsem0, gsem1, gsem2)
    ssem = (ssem0, ssem1, ssem2)
    esem = (esem0, esem1, esem2)

    @pl.when(s == 0)
    def _stage():
        pltpu.sync_copy(as_hbm, as_sh)
        pltpu.sync_copy(ad_hbm, ad_sh)
    pltpu.sync_copy(amax_hbm, amax_v)

    zf = jnp.zeros((L,), jnp.float32)

    def _zero_rows0(i, carry):
        for j in range(D // L):
            rows0[i, pl.ds(j * L, L)] = zf
        return carry
    lax.fori_loop(0, CH, _zero_rows0, 0)
    for g in range(CH // L):
        ea3[0, pl.ds(g * L, L)] = zf

    for k in range(SLAB // CH):
        pltpu.sync_copy(rows0, out_sh.at[pl.ds(s * SLAB + k * CH, CH)])
    rem = SLAB % CH
    pltpu.sync_copy(rows0.at[pl.ds(0, rem)],
                    out_sh.at[pl.ds(s * SLAB + SLAB - rem, rem)])
    for k in range(DEN_PAD // CH):
        pltpu.sync_copy(ea3.at[0], den_sh.at[pl.ds(s * DEN_PAD + k * CH, CH)])
    plsc.subcore_barrier()

    asmax = amax_v[pl.ds(0, L)][0]

    base = wid * NCHUNK

    def _start_idx(ci, b):
        pltpu.async_copy(sd_hbm.at[base + ci], idx3.at[b], isem[b])

    def _wait_idx(b):
        pltpu.make_async_copy(sd_hbm.at[0], idx3.at[b], isem[b]).wait()

    def _start_asad(b):
        pltpu.async_copy(as_sh.at[idx3.at[b, 0]], asb.at[b], asem[b])
        pltpu.async_copy(ad_sh.at[idx3.at[b, 1]], adb.at[b], bsem[b])

    def _wait_asad(b):
        pltpu.make_async_copy(as_sh.at[idx3.at[0, 0]], asb.at[b],
                              asem[b]).wait()
        pltpu.make_async_copy(ad_sh.at[idx3.at[0, 1]], adb.at[b],
                              bsem[b]).wait()

    def _start_gather(b):
        pltpu.async_copy(h_hbm.at[idx3.at[b, 0]], rows[b], gsem[b])

    def _wait_gather(b):
        pltpu.make_async_copy(h_hbm.at[idx3.at[0, 0]], rows[b],
                              gsem[b]).wait()

    def _start_scatter(b):
        pltpu.async_copy(ea3.at[b], den_sh.at[idx3.at[b, 1]], esem[b],
                         add=True)
        pltpu.async_copy(rows[b], out_sh.at[idx3.at[b, 1]], ssem[b],
                         add=True)

    def _wait_scatter(b):
        pltpu.make_async_copy(rows[b], out_sh.at[idx3.at[0, 1]],
                              ssem[b]).wait()
        pltpu.make_async_copy(ea3.at[b], den_sh.at[idx3.at[0, 1]],
                              esem[b]).wait()

    def _compute_ea(b):
        for g in range(CH // L):
            a_s = asb[b, pl.ds(g * L, L)]
            a_d = adb[b, pl.ds(g * L, L)]
            sa = a_s + a_d
            al = jnp.maximum(sa, 0.2 * sa)
            m0 = asmax + a_d
            m = jnp.maximum(m0, 0.2 * m0)
            ea3[b, pl.ds(g * L, L)] = jnp.exp(al - m)

    def _scale(b):
        rb = rows[b]

        def _body(g, carry):
            ea16 = ea3[b, pl.ds(g * L, L)]
            base_r = g * L
            for i in range(L):
                e = ea16.at[jnp.full((L,), i, jnp.int32)].get(
                    mode="promise_in_bounds")
                for j in range(D // L):
                    rb[base_r + i, pl.ds(j * L, L)] = (
                        rb[base_r + i, pl.ds(j * L, L)] * e)
            return carry
        lax.fori_loop(0, CH // L, _body, 0)

    def _phase(ci, b, first=False, fetch=True, prep=True):
        nb = (b + 1) % 3
        pb = (b + 2) % 3
        _wait_asad(b)
        _compute_ea(b)
        _wait_gather(b)
        if prep:
            _wait_idx(nb)
            _start_gather(nb)
            _start_asad(nb)
        _scale(b)
        _start_scatter(b)
        if not first:
            _wait_scatter(pb)
        if fetch:
            _start_idx(ci + 2, pb)

    _start_idx(0, 0)
    _start_idx(1, 1)
    _wait_idx(0)
    _start_asad(0)
    _start_gather(0)
    _phase(0, 0, first=True)

    def _loop_body(i, carry):
        ci = i * 3 + 1
        for p in range(3):
            _phase(ci + p, (1 + p) % 3)
        return carry

    lax.fori_loop(0, 40, _loop_body, 0)

    _phase(121, 1)
    _phase(122, 2)
    _phase(123, 0, fetch=False)
    _phase(124, 1, fetch=False, prep=False)
    _wait_scatter(1)

    plsc.subcore_barrier()
    pltpu.async_copy(out_sh.at[pl.ds(s * SLAB, SLAB)],
                     num_hbm.at[c, pl.ds(s * SLAB, SLAB)], isem0)
    pltpu.async_copy(den_sh.at[pl.ds(s * DEN_PAD, DEN_PAD)],
                     den_hbm.at[c, pl.ds(s * DEN_PAD, DEN_PAD)], isem1)
    pltpu.make_async_copy(out_sh.at[pl.ds(s * SLAB, SLAB)],
                          num_hbm.at[c, pl.ds(s * SLAB, SLAB)], isem0).wait()
    pltpu.make_async_copy(den_sh.at[pl.ds(s * DEN_PAD, DEN_PAD)],
                          den_hbm.at[c, pl.ds(s * DEN_PAD, DEN_PAD)],
                          isem1).wait()


_sc_edge = pl.kernel(
    _sc_body,
    out_type=[
        jax.ShapeDtypeStruct((NC, NPAD, D), jnp.float32),
        jax.ShapeDtypeStruct((NC, NS * DEN_PAD), jnp.float32),
    ],
    mesh=plsc.VectorSubcoreMesh(core_axis_name="c", subcore_axis_name="s"),
    compiler_params=pltpu.CompilerParams(needs_layout_passes=False),
    scratch_types=[
        pltpu.VMEM((L,), jnp.float32),
        pltpu.VMEM((3, 2, CH), jnp.int32),
        pltpu.VMEM((3, CH), jnp.float32),
        pltpu.VMEM((3, CH), jnp.float32),
        pltpu.VMEM((3, CH), jnp.float32),
        pltpu.VMEM((CH, D), jnp.float32),
        pltpu.VMEM((CH, D), jnp.float32),
        pltpu.VMEM((CH, D), jnp.float32),
        pltpu.VMEM_SHARED((N,), jnp.float32),
        pltpu.VMEM_SHARED((N,), jnp.float32),
        pltpu.VMEM_SHARED((NPAD, D), jnp.float32),
        pltpu.VMEM_SHARED((NS * DEN_PAD,), jnp.float32),
        pltpu.SemaphoreType.DMA,
        pltpu.SemaphoreType.DMA,
        pltpu.SemaphoreType.DMA,
        pltpu.SemaphoreType.DMA,
        pltpu.SemaphoreType.DMA,
        pltpu.SemaphoreType.DMA,
        pltpu.SemaphoreType.DMA,
        pltpu.SemaphoreType.DMA,
        pltpu.SemaphoreType.DMA,
        pltpu.SemaphoreType.DMA,
        pltpu.SemaphoreType.DMA,
        pltpu.SemaphoreType.DMA,
        pltpu.SemaphoreType.DMA,
        pltpu.SemaphoreType.DMA,
        pltpu.SemaphoreType.DMA,
        pltpu.SemaphoreType.DMA,
        pltpu.SemaphoreType.DMA,
        pltpu.SemaphoreType.DMA,
    ],
)


def kernel(x, edge_index, W0, a_src0, a_dst0, b0, W1, a_src1, a_dst1, b1,
           W2, a_src2, a_dst2, b2):
    src3 = edge_index[0].reshape(NW, NCHUNK, CH)
    dst3 = edge_index[1].reshape(NW, NCHUNK, CH)
    sd = jnp.stack([src3, dst3], axis=2).reshape(NW * NCHUNK, 2, CH)

    h0, as0, ad0, am0 = _proj(x, W0, a_src0, a_dst0)
    num0, den0 = _sc_edge(h0, as0, ad0, am0, sd)
    h1, as1, ad1, am1 = _comb_proj(num0, den0, h0, as0, ad0, b0,
                                   W1, a_src1, a_dst1)
    num1, den1 = _sc_edge(h1, as1, ad1, am1, sd)
    h2, as2, ad2, am2 = _comb_proj(num1, den1, h1, as1, ad1, b1,
                                   W2, a_src2, a_dst2)
    num2, den2 = _sc_edge(h2, as2, ad2, am2, sd)
    (out,) = _final(num2, den2, h2, as2, ad2, b2)
    return out
